# Initial kernel scaffold; baseline (speedup 1.0000x reference)
import jax
import jax.numpy as jnp
from jax.experimental import pallas as pl


def kernel(x, edge_index, edge_weight, mask, emb_table, egn_w, egn_b, egn_ms, mid_gn_w, mid_gn_b, mid_gn_ms, out_gn_w, out_gn_b, out_gn_ms, l1_Wt0, l1_Wt1, l1_bt0, l1_bt1, l1_gn_w, l1_gn_b, l1_gn_ms, l1_Wc0, l1_Wc1, l1_bc0, l1_bc1, l2_Wt0, l2_Wt1, l2_bt0, l2_bt1, l2_gn_w, l2_gn_b, l2_gn_ms, l2_Wc0, l2_Wc1, l2_bc0, l2_bc1):
    raise NotImplementedError("write your pallas kernel here")


# speedup vs baseline: 2.8229x; 2.8229x over previous
import functools

import jax
import jax.numpy as jnp
from jax import lax
from jax.experimental import pallas as pl
from jax.experimental.pallas import tpu as pltpu
from jax.experimental.pallas import tpu_sc as plsc

N = 50000
E = 800000
H = 64
VOCAB = 64
Z = 0.8
EPS = 1e-5

NS = 16
K = 128
E_PAD = 802816
CH = E_PAD // (NS * K)
HALF = N // 2
ACC_ROWS = 26624
ZROWS = ACC_ROWS // NS
ZCH = ZROWS // K
OUT_REM = HALF - 15 * ZROWS

R_BLK = 2000
NB = N // R_BLK


def _sc_mesh():
    return plsc.VectorSubcoreMesh(core_axis_name="c", subcore_axis_name="s")


def _sc_deg(row_p, w_p):
    @functools.partial(
        pl.kernel,
        out_type=jax.ShapeDtypeStruct((N,), jnp.float32),
        mesh=_sc_mesh(),
        scratch_types=[
            pltpu.VMEM_SHARED((ACC_ROWS,), jnp.float32),
            pltpu.VMEM((K,), jnp.int32),
            pltpu.VMEM((K,), jnp.int32),
            pltpu.VMEM((K,), jnp.float32),
        ],
    )
    def k(row_hbm, w_hbm, out_hbm, accd, row_v, sidx_v, w_v):
        c = lax.axis_index("c")
        s = lax.axis_index("s")
        for j in range(K // 16):
            w_v[pl.ds(j * 16, 16)] = jnp.zeros((16,), jnp.float32)

        def zbody(t, carry):
            pltpu.sync_copy(w_v, accd.at[pl.ds(s * ZROWS + t * K, K)])
            return carry

        lax.fori_loop(0, ZCH, zbody, 0)
        plsc.subcore_barrier()
        base = c * HALF

        def body(ch, carry):
            e0 = (s * CH + ch) * K
            pltpu.sync_copy(row_hbm.at[pl.ds(e0, K)], row_v)
            pltpu.sync_copy(w_hbm.at[pl.ds(e0, K)], w_v)
            for j in range(K // 16):
                r = row_v[pl.ds(j * 16, 16)]
                loc = r - base
                ok = (loc >= 0) & (loc < HALF)
                sidx_v[pl.ds(j * 16, 16)] = jnp.where(ok, loc, HALF)
            pltpu.sync_copy(w_v, accd.at[sidx_v], add=True)
            return carry

        lax.fori_loop(0, CH, body, 0)
        plsc.subcore_barrier()
        ob = c * HALF

        @pl.when(s < NS - 1)
        def _():
            def obody(t, carry):
                off = s * ZROWS + t * K
                pltpu.sync_copy(accd.at[pl.ds(off, K)], w_v)
                pltpu.sync_copy(w_v, out_hbm.at[pl.ds(ob + off, K)])
                return carry

            lax.fori_loop(0, ZCH, obody, 0)

        @pl.when(s == NS - 1)
        def _():
            off = 15 * ZROWS
            pltpu.sync_copy(accd.at[pl.ds(off, OUT_REM)],
                            w_v.at[pl.ds(0, OUT_REM)])
            pltpu.sync_copy(w_v.at[pl.ds(0, OUT_REM)],
                            out_hbm.at[pl.ds(ob + off, OUT_REM)])

    return k(row_p, w_p)


def _sc_msg(x, col_p, row_p, w_p):
    @functools.partial(
        pl.kernel,
        out_type=jax.ShapeDtypeStruct((N, H), jnp.float32),
        mesh=_sc_mesh(),
        compiler_params=pltpu.CompilerParams(use_tc_tiling_on_sc=False),
        scratch_types=[
            pltpu.VMEM_SHARED((ACC_ROWS, H), jnp.float32),
            pltpu.VMEM((K,), jnp.int32),
            pltpu.VMEM((K,), jnp.int32),
            pltpu.VMEM((K,), jnp.int32),
            pltpu.VMEM((K,), jnp.float32),
            pltpu.VMEM((K, H), jnp.float32),
            pltpu.SemaphoreType.DMA,
        ],
    )
    def k(x_hbm, col_hbm, row_hbm, w_hbm, out_hbm,
          acc, idx_v, row_v, sidx_v, w_v, rows_v, sem):
        c = lax.axis_index("c")
        s = lax.axis_index("s")

        def zrbody(e, carry):
            for j in range(H // 16):
                rows_v[e, pl.ds(j * 16, 16)] = jnp.zeros((16,), jnp.float32)
            return carry

        lax.fori_loop(0, K, zrbody, 0)

        def zbody(t, carry):
            pltpu.sync_copy(rows_v, acc.at[pl.ds(s * ZROWS + t * K, K)])
            return carry

        lax.fori_loop(0, ZCH, zbody, 0)
        plsc.subcore_barrier()
        base = c * HALF

        def body(ch, carry):
            e0 = (s * CH + ch) * K
            pltpu.sync_copy(col_hbm.at[pl.ds(e0, K)], idx_v)
            pltpu.sync_copy(row_hbm.at[pl.ds(e0, K)], row_v)
            pltpu.sync_copy(w_hbm.at[pl.ds(e0, K)], w_v)
            for j in range(K // 16):
                r = row_v[pl.ds(j * 16, 16)]
                loc = r - base
                ok = (loc >= 0) & (loc < HALF)
                sidx_v[pl.ds(j * 16, 16)] = jnp.where(ok, loc, HALF)
            pltpu.async_copy(x_hbm.at[idx_v], rows_v, sem).wait()

            def sbody(g, cc):
                wv16 = w_v[pl.ds(g * 16, 16)]
                for l in range(16):
                    wl = wv16[l]
                    e = g * 16 + l
                    for j in range(H // 16):
                        rows_v[e, pl.ds(j * 16, 16)] = (
                            rows_v[e, pl.ds(j * 16, 16)] * wl)
                return cc

            lax.fori_loop(0, K // 16, sbody, 0)
            pltpu.sync_copy(rows_v, acc.at[sidx_v], add=True)
            return carry

        lax.fori_loop(0, CH, body, 0)
        plsc.subcore_barrier()
        ob = c * HALF

        @pl.when(s < NS - 1)
        def _():
            def obody(t, carry):
                off = s * ZROWS + t * K
                pltpu.sync_copy(acc.at[pl.ds(off, K)], rows_v)
                pltpu.sync_copy(rows_v, out_hbm.at[pl.ds(ob + off, K)])
                return carry

            lax.fori_loop(0, ZCH, obody, 0)

        @pl.when(s == NS - 1)
        def _():
            off = 15 * ZROWS
            pltpu.sync_copy(acc.at[pl.ds(off, OUT_REM)],
                            rows_v.at[pl.ds(0, OUT_REM)])
            pltpu.sync_copy(rows_v.at[pl.ds(0, OUT_REM)],
                            out_hbm.at[pl.ds(ob + off, OUT_REM)])

    return k(x, col_p, row_p, w_p)


def _rows(w):
    return pl.BlockSpec((R_BLK, w), lambda i: (i, 0))


def _full(shape):
    nd = len(shape)
    return pl.BlockSpec(shape, lambda i: (0,) * nd)


def _stats_update(out_ref, vals):
    i = pl.program_id(0)
    sm = jnp.sum(vals, axis=0, keepdims=True)
    sq = jnp.sum(vals * vals, axis=0, keepdims=True)
    upd = jnp.concatenate(
        [sm, sq, jnp.zeros((6, H), jnp.float32)], axis=0)

    @pl.when(i == 0)
    def _():
        out_ref[...] = jnp.zeros_like(out_ref)

    out_ref[...] += upd


def _norm(xv, st_ref, w, b, ms):
    st = st_ref[...]
    mean = st[0:1, :] * (1.0 / N)
    ex2 = st[1:2, :] * (1.0 / N)
    var = ex2 - (2.0 - ms) * ms * mean * mean
    return w * (xv - ms * mean) * lax.rsqrt(var + EPS) + b


def _blend(x0, x1, mf):
    return mf * (Z * x1 + (1.0 - Z) * x0) + (1.0 - mf) * (Z * x0 + (1.0 - Z) * x1)


def _onehot_emb(ids_ref, emb_ref):
    ids = ids_ref[...]
    oh = (ids == lax.broadcasted_iota(jnp.int32, (R_BLK, VOCAB), 1))
    return jnp.dot(oh.astype(jnp.float32), emb_ref[...],
                   preferred_element_type=jnp.float32)


def _k_emb_stats(ids_ref, emb_ref, out_ref):
    _stats_update(out_ref, _onehot_emb(ids_ref, emb_ref))


def _emb_stats(ids, emb):
    return pl.pallas_call(
        _k_emb_stats,
        grid=(NB,),
        in_specs=[_rows(1), _full((VOCAB, H))],
        out_specs=_full((8, H)),
        out_shape=jax.ShapeDtypeStruct((8, H), jnp.float32),
    )(ids, emb)


def _k_emb_apply(ids_ref, emb_ref, st_ref, gw_ref, gb_ref, gms_ref, mf_ref,
                 wt0_ref, wt1_ref, bt0_ref, bt1_ref, h_ref, xa_ref):
    h0 = _onehot_emb(ids_ref, emb_ref)
    h = _norm(h0, st_ref, gw_ref[...], gb_ref[...], gms_ref[...])
    mf = mf_ref[...]
    x1 = jnp.maximum(jnp.dot(h, wt1_ref[...],
                             preferred_element_type=jnp.float32) + bt1_ref[...], 0.0)
    x0 = jnp.maximum(jnp.dot(h, wt0_ref[...],
                             preferred_element_type=jnp.float32) + bt0_ref[...], 0.0)
    h_ref[...] = h
    xa_ref[...] = _blend(x0, x1, mf)


def _emb_apply(ids, emb, st, gw, gb, gms, mf, wt0, wt1, bt0, bt1):
    return pl.pallas_call(
        _k_emb_apply,
        grid=(NB,),
        in_specs=[_rows(1), _full((VOCAB, H)), _full((8, H)),
                  _full((1, H)), _full((1, H)), _full((1, H)), _rows(1),
                  _full((H, H)), _full((H, H)), _full((1, H)), _full((1, H))],
        out_specs=[_rows(H), _rows(H)],
        out_shape=[jax.ShapeDtypeStruct((N, H), jnp.float32),
                   jax.ShapeDtypeStruct((N, H), jnp.float32)],
    )(ids, emb, st, gw, gb, gms, mf, wt0, wt1, bt0, bt1)


def _k_agg_stats(agg_ref, deg_ref, out_ref):
    deg = deg_ref[...]
    degf = jnp.where(deg < 0.5, deg + 1.0, deg)
    _stats_update(out_ref, agg_ref[...] / degf)


def _agg_stats(agg, deg):
    return pl.pallas_call(
        _k_agg_stats,
        grid=(NB,),
        in_specs=[_rows(H), _rows(1)],
        out_specs=_full((8, H)),
        out_shape=jax.ShapeDtypeStruct((8, H), jnp.float32),
    )(agg, deg)


def _k_conv_apply(agg_ref, deg_ref, st_ref, gw_ref, gb_ref, gms_ref,
                  hprev_ref, mf_ref, wc0_ref, wc1_ref, bc0_ref, bc1_ref,
                  h2_ref, st2_ref):
    deg = deg_ref[...]
    degf = jnp.where(deg < 0.5, deg + 1.0, deg)
    y = agg_ref[...] / degf
    yn = _norm(y, st_ref, gw_ref[...], gb_ref[...], gms_ref[...])
    cat = jnp.concatenate([yn, hprev_ref[...]], axis=1)
    c1 = jnp.dot(cat, wc1_ref[...], preferred_element_type=jnp.float32) + bc1_ref[...]
    c0 = jnp.dot(cat, wc0_ref[...], preferred_element_type=jnp.float32) + bc0_ref[...]
    h2 = _blend(c0, c1, mf_ref[...])
    h2_ref[...] = h2
    _stats_update(st2_ref, h2)


def _conv_apply(agg, deg, st, gw, gb, gms, hprev, mf, wc0, wc1, bc0, bc1):
    return pl.pallas_call(
        _k_conv_apply,
        grid=(NB,),
        in_specs=[_rows(H), _rows(1), _full((8, H)),
                  _full((1, H)), _full((1, H)), _full((1, H)),
                  _rows(H), _rows(1),
                  _full((2 * H, H)), _full((2 * H, H)),
                  _full((1, H)), _full((1, H))],
        out_specs=[_rows(H), _full((8, H))],
        out_shape=[jax.ShapeDtypeStruct((N, H), jnp.float32),
                   jax.ShapeDtypeStruct((8, H), jnp.float32)],
    )(agg, deg, st, gw, gb, gms, hprev, mf, wc0, wc1, bc0, bc1)


def _k_mid_apply(h2_ref, st_ref, gw_ref, gb_ref, gms_ref, mf_ref,
                 wt0_ref, wt1_ref, bt0_ref, bt1_ref, hp_ref, xb_ref):
    hp = jnp.maximum(
        _norm(h2_ref[...], st_ref, gw_ref[...], gb_ref[...], gms_ref[...]), 0.0)
    mf = mf_ref[...]
    x1 = jnp.maximum(jnp.dot(hp, wt1_ref[...],
                             preferred_element_type=jnp.float32) + bt1_ref[...], 0.0)
    x0 = jnp.maximum(jnp.dot(hp, wt0_ref[...],
                             preferred_element_type=jnp.float32) + bt0_ref[...], 0.0)
    hp_ref[...] = hp
    xb_ref[...] = _blend(x0, x1, mf)


def _mid_apply(h2, st, gw, gb, gms, mf, wt0, wt1, bt0, bt1):
    return pl.pallas_call(
        _k_mid_apply,
        grid=(NB,),
        in_specs=[_rows(H), _full((8, H)),
                  _full((1, H)), _full((1, H)), _full((1, H)), _rows(1),
                  _full((H, H)), _full((H, H)), _full((1, H)), _full((1, H))],
        out_specs=[_rows(H), _rows(H)],
        out_shape=[jax.ShapeDtypeStruct((N, H), jnp.float32),
                   jax.ShapeDtypeStruct((N, H), jnp.float32)],
    )(h2, st, gw, gb, gms, mf, wt0, wt1, bt0, bt1)


def _k_out_apply(z_ref, st_ref, gw_ref, gb_ref, gms_ref, out_ref):
    out_ref[...] = _norm(z_ref[...], st_ref, gw_ref[...], gb_ref[...], gms_ref[...])


def _out_apply(z, st, gw, gb, gms):
    return pl.pallas_call(
        _k_out_apply,
        grid=(NB,),
        in_specs=[_rows(H), _full((8, H)),
                  _full((1, H)), _full((1, H)), _full((1, H))],
        out_specs=_rows(H),
        out_shape=jax.ShapeDtypeStruct((N, H), jnp.float32),
    )(z, st, gw, gb, gms)


def kernel(x, edge_index, edge_weight, mask, emb_table,
           egn_w, egn_b, egn_ms, mid_gn_w, mid_gn_b, mid_gn_ms,
           out_gn_w, out_gn_b, out_gn_ms,
           l1_Wt0, l1_Wt1, l1_bt0, l1_bt1, l1_gn_w, l1_gn_b, l1_gn_ms,
           l1_Wc0, l1_Wc1, l1_bc0, l1_bc1,
           l2_Wt0, l2_Wt1, l2_bt0, l2_bt1, l2_gn_w, l2_gn_b, l2_gn_ms,
           l2_Wc0, l2_Wc1, l2_bc0, l2_bc1):
    ids = x.reshape(N, 1).astype(jnp.int32)
    row = edge_index[0].astype(jnp.int32)
    col = edge_index[1].astype(jnp.int32)
    pad = E_PAD - E
    row_p = jnp.concatenate([row, jnp.full((pad,), N, jnp.int32)])
    col_p = jnp.concatenate([col, jnp.zeros((pad,), jnp.int32)])
    w_p = jnp.concatenate([edge_weight.astype(jnp.float32),
                           jnp.zeros((pad,), jnp.float32)])
    mf = mask.astype(jnp.float32)

    r1 = lambda a: a.reshape(1, H)

    deg = _sc_deg(row_p, w_p).reshape(N, 1)

    st0 = _emb_stats(ids, emb_table)
    h, xa = _emb_apply(ids, emb_table, st0, r1(egn_w), r1(egn_b), r1(egn_ms),
                       mf, l1_Wt0, l1_Wt1, r1(l1_bt0), r1(l1_bt1))

    agg1 = _sc_msg(xa, col_p, row_p, w_p)
    st1 = _agg_stats(agg1, deg)
    h2, st2 = _conv_apply(agg1, deg, st1, r1(l1_gn_w), r1(l1_gn_b),
                          r1(l1_gn_ms), h, mf, l1_Wc0, l1_Wc1,
                          r1(l1_bc0), r1(l1_bc1))
    hp, xb = _mid_apply(h2, st2, r1(mid_gn_w), r1(mid_gn_b), r1(mid_gn_ms),
                        mf, l2_Wt0, l2_Wt1, r1(l2_bt0), r1(l2_bt1))

    agg2 = _sc_msg(xb, col_p, row_p, w_p)
    st3 = _agg_stats(agg2, deg)
    zz, st4 = _conv_apply(agg2, deg, st3, r1(l2_gn_w), r1(l2_gn_b),
                          r1(l2_gn_ms), hp, mf, l2_Wc0, l2_Wc1,
                          r1(l2_bc0), r1(l2_bc1))
    return _out_apply(zz, st4, r1(out_gn_w), r1(out_gn_b), r1(out_gn_ms))


# speedup vs baseline: 3.9588x; 1.4024x over previous
import functools

import jax
import jax.numpy as jnp
from jax import lax
from jax.experimental import pallas as pl
from jax.experimental.pallas import tpu as pltpu
from jax.experimental.pallas import tpu_sc as plsc

N = 50000
E = 800000
H = 64
VOCAB = 64
Z = 0.8
EPS = 1e-5

NS = 16
K = 128
E_PAD = 802816
CH = E_PAD // (NS * K)
HALF = N // 2
ACC_ROWS = 26624
ZROWS = ACC_ROWS // NS
ZCH = ZROWS // K
OUT_REM = HALF - 15 * ZROWS
NCHUNK = E_PAD // K
MACC = 25088
MZROWS = MACC // NS
MZCH = MZROWS // K
MZREM = MZROWS - MZCH * K
MOCH15 = (HALF - 15 * MZROWS) // K
MOREM15 = HALF - 15 * MZROWS - MOCH15 * K

R_BLK = 2000
NB = N // R_BLK


def _sc_mesh():
    return plsc.VectorSubcoreMesh(core_axis_name="c", subcore_axis_name="s")


def _sc_deg(row_p, w_p):
    @functools.partial(
        pl.kernel,
        out_type=jax.ShapeDtypeStruct((N,), jnp.float32),
        mesh=_sc_mesh(),
        scratch_types=[
            pltpu.VMEM_SHARED((ACC_ROWS,), jnp.float32),
            pltpu.VMEM((K,), jnp.int32),
            pltpu.VMEM((K,), jnp.int32),
            pltpu.VMEM((K,), jnp.float32),
        ],
    )
    def k(row_hbm, w_hbm, out_hbm, accd, row_v, sidx_v, w_v):
        c = lax.axis_index("c")
        s = lax.axis_index("s")
        for j in range(K // 16):
            w_v[pl.ds(j * 16, 16)] = jnp.zeros((16,), jnp.float32)

        def zbody(t, carry):
            pltpu.sync_copy(w_v, accd.at[pl.ds(s * ZROWS + t * K, K)])
            return carry

        lax.fori_loop(0, ZCH, zbody, 0)
        plsc.subcore_barrier()
        base = c * HALF

        def body(ch, carry):
            e0 = (s * CH + ch) * K
            pltpu.sync_copy(row_hbm.at[pl.ds(e0, K)], row_v)
            pltpu.sync_copy(w_hbm.at[pl.ds(e0, K)], w_v)
            for j in range(K // 16):
                r = row_v[pl.ds(j * 16, 16)]
                loc = r - base
                ok = (loc >= 0) & (loc < HALF)
                sidx_v[pl.ds(j * 16, 16)] = jnp.where(ok, loc, HALF)
            pltpu.sync_copy(w_v, accd.at[sidx_v], add=True)
            return carry

        lax.fori_loop(0, CH, body, 0)
        plsc.subcore_barrier()
        ob = c * HALF

        @pl.when(s < NS - 1)
        def _():
            def obody(t, carry):
                off = s * ZROWS + t * K
                pltpu.sync_copy(accd.at[pl.ds(off, K)], w_v)
                pltpu.sync_copy(w_v, out_hbm.at[pl.ds(ob + off, K)])
                return carry

            lax.fori_loop(0, ZCH, obody, 0)

        @pl.when(s == NS - 1)
        def _():
            off = 15 * ZROWS
            pltpu.sync_copy(accd.at[pl.ds(off, OUT_REM)],
                            w_v.at[pl.ds(0, OUT_REM)])
            pltpu.sync_copy(w_v.at[pl.ds(0, OUT_REM)],
                            out_hbm.at[pl.ds(ob + off, OUT_REM)])

    return k(row_p, w_p)


def _sc_msg(x, ed_p, w_p):
    @functools.partial(
        pl.kernel,
        out_type=jax.ShapeDtypeStruct((N, H), jnp.float32),
        mesh=_sc_mesh(),
        compiler_params=pltpu.CompilerParams(use_tc_tiling_on_sc=False),
        scratch_types=[
            pltpu.VMEM_SHARED((MACC, H), jnp.float32),
            pltpu.VMEM((2, K), jnp.int32),
            pltpu.VMEM((2, K), jnp.int32),
            pltpu.VMEM((K,), jnp.float32),
            pltpu.VMEM((K,), jnp.float32),
            pltpu.VMEM((1, K), jnp.int32),
            pltpu.VMEM((1, K), jnp.int32),
            pltpu.VMEM((K, H), jnp.float32),
            pltpu.VMEM((K, H), jnp.float32),
            pltpu.SemaphoreType.DMA,
            pltpu.SemaphoreType.DMA,
            pltpu.SemaphoreType.DMA,
            pltpu.SemaphoreType.DMA,
        ],
    )
    def k(x_hbm, ed_hbm, w_hbm, out_hbm,
          acc, eb0, eb1, wb0, wb1, si0, si1, rb0, rb1,
          gs0, gs1, es0, es1):
        c = lax.axis_index("c")
        s = lax.axis_index("s")
        eb = (eb0, eb1)
        wb = (wb0, wb1)
        si = (si0, si1)
        rb = (rb0, rb1)
        gs = (gs0, gs1)
        es = (es0, es1)

        def zrbody(e, carry):
            for j in range(H // 16):
                rb0[e, pl.ds(j * 16, 16)] = jnp.zeros((16,), jnp.float32)
            return carry

        lax.fori_loop(0, K, zrbody, 0)

        def zbody(t, carry):
            pltpu.sync_copy(rb0, acc.at[pl.ds(s * MZROWS + t * K, K)])
            return carry

        lax.fori_loop(0, MZCH, zbody, 0)
        pltpu.sync_copy(rb0.at[pl.ds(0, MZREM)],
                        acc.at[pl.ds(s * MZROWS + MZCH * K, MZREM)])
        plsc.subcore_barrier()
        base = c * HALF
        ch0 = s * CH

        def compute_sidx(ebuf, wbuf, sbuf):
            for g in range(K // 16):
                r = ebuf[1, pl.ds(g * 16, 16)]
                loc = r - base
                ok = (loc >= 0) & (loc < HALF)
                sbuf[0, pl.ds(g * 16, 16)] = jnp.where(ok, loc, 0)
                w16 = wbuf[pl.ds(g * 16, 16)]
                wbuf[pl.ds(g * 16, 16)] = jnp.where(
                    ok, w16, jnp.zeros((16,), jnp.float32))

        def start_gather(ebuf, rbuf, sem):
            pltpu.async_copy(x_hbm.at[ebuf.at[0]], rbuf, sem)

        def drain_gather(ebuf, rbuf, sem):
            pltpu.make_async_copy(x_hbm.at[ebuf.at[0]], rbuf, sem).wait()

        def start_edload(S, ebuf, wbuf, sem):
            pltpu.async_copy(ed_hbm.at[ch0 + S], ebuf, sem)
            pltpu.async_copy(w_hbm.at[pl.ds((ch0 + S) * K, K)], wbuf, sem)

        def drain_edload(S, ebuf, wbuf, sem):
            pltpu.make_async_copy(ed_hbm.at[ch0 + S],
                                  ebuf, sem).wait()
            pltpu.make_async_copy(w_hbm.at[pl.ds((ch0 + S) * K, K)],
                                  wbuf, sem).wait()

        def scale_scatter(wbuf, sbuf, rbuf):
            def sgrp(g, cc):
                wv16 = wbuf[pl.ds(g * 16, 16)]
                for l in range(16):
                    wl = wv16[l]
                    e = g * 16 + l
                    for q in range(H // 16):
                        rbuf[e, pl.ds(q * 16, 16)] = (
                            rbuf[e, pl.ds(q * 16, 16)] * wl)
                return cc

            lax.fori_loop(0, K // 16, sgrp, 0)
            pltpu.sync_copy(rbuf, acc.at[sbuf.at[0]], add=True)

        start_edload(0, eb0, wb0, es0)
        drain_edload(0, eb0, wb0, es0)
        compute_sidx(eb0, wb0, si0)
        start_gather(eb0, rb0, gs0)
        start_edload(1, eb1, wb1, es1)

        def phase(S, p):
            drain_gather(eb[p], rb[p], gs[p])

            @pl.when(S < CH - 1)
            def _():
                drain_edload(S + 1, eb[1 - p], wb[1 - p], es[1 - p])
                compute_sidx(eb[1 - p], wb[1 - p], si[1 - p])
                start_gather(eb[1 - p], rb[1 - p], gs[1 - p])

            scale_scatter(wb[p], si[p], rb[p])

            @pl.when(S < CH - 2)
            def _():
                start_edload(S + 2, eb[p], wb[p], es[p])

        def body(t, carry):
            phase(2 * t, 0)
            phase(2 * t + 1, 1)
            return carry

        lax.fori_loop(0, CH // 2, body, 0)
        plsc.subcore_barrier()
        ob = c * HALF

        def ocopy(off, rows):
            pltpu.sync_copy(acc.at[pl.ds(off, rows)],
                            rb0.at[pl.ds(0, rows)])
            pltpu.sync_copy(rb0.at[pl.ds(0, rows)],
                            out_hbm.at[pl.ds(ob + off, rows)])

        @pl.when(s < NS - 1)
        def _():
            def obody(t, carry):
                ocopy(s * MZROWS + t * K, K)
                return carry

            lax.fori_loop(0, MZCH, obody, 0)
            ocopy(s * MZROWS + MZCH * K, MZREM)

        @pl.when(s == NS - 1)
        def _():
            def obody(t, carry):
                ocopy(15 * MZROWS + t * K, K)
                return carry

            lax.fori_loop(0, MOCH15, obody, 0)
            ocopy(15 * MZROWS + MOCH15 * K, MOREM15)

    return k(x, ed_p, w_p)


def _rows(w):
    return pl.BlockSpec((R_BLK, w), lambda i: (i, 0))


def _full(shape):
    nd = len(shape)
    return pl.BlockSpec(shape, lambda i: (0,) * nd)


def _stats_update(out_ref, vals):
    i = pl.program_id(0)
    sm = jnp.sum(vals, axis=0, keepdims=True)
    sq = jnp.sum(vals * vals, axis=0, keepdims=True)
    upd = jnp.concatenate(
        [sm, sq, jnp.zeros((6, H), jnp.float32)], axis=0)

    @pl.when(i == 0)
    def _():
        out_ref[...] = jnp.zeros_like(out_ref)

    out_ref[...] += upd


def _norm(xv, st_ref, w, b, ms):
    st = st_ref[...]
    mean = st[0:1, :] * (1.0 / N)
    ex2 = st[1:2, :] * (1.0 / N)
    var = ex2 - (2.0 - ms) * ms * mean * mean
    return w * (xv - ms * mean) * lax.rsqrt(var + EPS) + b


def _blend(x0, x1, mf):
    return mf * (Z * x1 + (1.0 - Z) * x0) + (1.0 - mf) * (Z * x0 + (1.0 - Z) * x1)


def _onehot_emb(ids_ref, emb_ref):
    ids = ids_ref[...]
    oh = (ids == lax.broadcasted_iota(jnp.int32, (R_BLK, VOCAB), 1))
    return jnp.dot(oh.astype(jnp.float32), emb_ref[...],
                   preferred_element_type=jnp.float32)


def _k_emb_stats(ids_ref, emb_ref, out_ref):
    _stats_update(out_ref, _onehot_emb(ids_ref, emb_ref))


def _emb_stats(ids, emb):
    return pl.pallas_call(
        _k_emb_stats,
        grid=(NB,),
        in_specs=[_rows(1), _full((VOCAB, H))],
        out_specs=_full((8, H)),
        out_shape=jax.ShapeDtypeStruct((8, H), jnp.float32),
    )(ids, emb)


def _k_emb_apply(ids_ref, emb_ref, st_ref, gw_ref, gb_ref, gms_ref, mf_ref,
                 wt0_ref, wt1_ref, bt0_ref, bt1_ref, h_ref, xa_ref):
    h0 = _onehot_emb(ids_ref, emb_ref)
    h = _norm(h0, st_ref, gw_ref[...], gb_ref[...], gms_ref[...])
    mf = mf_ref[...]
    x1 = jnp.maximum(jnp.dot(h, wt1_ref[...],
                             preferred_element_type=jnp.float32) + bt1_ref[...], 0.0)
    x0 = jnp.maximum(jnp.dot(h, wt0_ref[...],
                             preferred_element_type=jnp.float32) + bt0_ref[...], 0.0)
    h_ref[...] = h
    xa_ref[...] = _blend(x0, x1, mf)


def _emb_apply(ids, emb, st, gw, gb, gms, mf, wt0, wt1, bt0, bt1):
    return pl.pallas_call(
        _k_emb_apply,
        grid=(NB,),
        in_specs=[_rows(1), _full((VOCAB, H)), _full((8, H)),
                  _full((1, H)), _full((1, H)), _full((1, H)), _rows(1),
                  _full((H, H)), _full((H, H)), _full((1, H)), _full((1, H))],
        out_specs=[_rows(H), _rows(H)],
        out_shape=[jax.ShapeDtypeStruct((N, H), jnp.float32),
                   jax.ShapeDtypeStruct((N, H), jnp.float32)],
    )(ids, emb, st, gw, gb, gms, mf, wt0, wt1, bt0, bt1)


def _k_agg_stats(agg_ref, deg_ref, out_ref):
    deg = deg_ref[...]
    degf = jnp.where(deg < 0.5, deg + 1.0, deg)
    _stats_update(out_ref, agg_ref[...] / degf)


def _agg_stats(agg, deg):
    return pl.pallas_call(
        _k_agg_stats,
        grid=(NB,),
        in_specs=[_rows(H), _rows(1)],
        out_specs=_full((8, H)),
        out_shape=jax.ShapeDtypeStruct((8, H), jnp.float32),
    )(agg, deg)


def _k_conv_apply(agg_ref, deg_ref, st_ref, gw_ref, gb_ref, gms_ref,
                  hprev_ref, mf_ref, wc0_ref, wc1_ref, bc0_ref, bc1_ref,
                  h2_ref, st2_ref):
    deg = deg_ref[...]
    degf = jnp.where(deg < 0.5, deg + 1.0, deg)
    y = agg_ref[...] / degf
    yn = _norm(y, st_ref, gw_ref[...], gb_ref[...], gms_ref[...])
    cat = jnp.concatenate([yn, hprev_ref[...]], axis=1)
    c1 = jnp.dot(cat, wc1_ref[...], preferred_element_type=jnp.float32) + bc1_ref[...]
    c0 = jnp.dot(cat, wc0_ref[...], preferred_element_type=jnp.float32) + bc0_ref[...]
    h2 = _blend(c0, c1, mf_ref[...])
    h2_ref[...] = h2
    _stats_update(st2_ref, h2)


def _conv_apply(agg, deg, st, gw, gb, gms, hprev, mf, wc0, wc1, bc0, bc1):
    return pl.pallas_call(
        _k_conv_apply,
        grid=(NB,),
        in_specs=[_rows(H), _rows(1), _full((8, H)),
                  _full((1, H)), _full((1, H)), _full((1, H)),
                  _rows(H), _rows(1),
                  _full((2 * H, H)), _full((2 * H, H)),
                  _full((1, H)), _full((1, H))],
        out_specs=[_rows(H), _full((8, H))],
        out_shape=[jax.ShapeDtypeStruct((N, H), jnp.float32),
                   jax.ShapeDtypeStruct((8, H), jnp.float32)],
    )(agg, deg, st, gw, gb, gms, hprev, mf, wc0, wc1, bc0, bc1)


def _k_mid_apply(h2_ref, st_ref, gw_ref, gb_ref, gms_ref, mf_ref,
                 wt0_ref, wt1_ref, bt0_ref, bt1_ref, hp_ref, xb_ref):
    hp = jnp.maximum(
        _norm(h2_ref[...], st_ref, gw_ref[...], gb_ref[...], gms_ref[...]), 0.0)
    mf = mf_ref[...]
    x1 = jnp.maximum(jnp.dot(hp, wt1_ref[...],
                             preferred_element_type=jnp.float32) + bt1_ref[...], 0.0)
    x0 = jnp.maximum(jnp.dot(hp, wt0_ref[...],
                             preferred_element_type=jnp.float32) + bt0_ref[...], 0.0)
    hp_ref[...] = hp
    xb_ref[...] = _blend(x0, x1, mf)


def _mid_apply(h2, st, gw, gb, gms, mf, wt0, wt1, bt0, bt1):
    return pl.pallas_call(
        _k_mid_apply,
        grid=(NB,),
        in_specs=[_rows(H), _full((8, H)),
                  _full((1, H)), _full((1, H)), _full((1, H)), _rows(1),
                  _full((H, H)), _full((H, H)), _full((1, H)), _full((1, H))],
        out_specs=[_rows(H), _rows(H)],
        out_shape=[jax.ShapeDtypeStruct((N, H), jnp.float32),
                   jax.ShapeDtypeStruct((N, H), jnp.float32)],
    )(h2, st, gw, gb, gms, mf, wt0, wt1, bt0, bt1)


def _k_out_apply(z_ref, st_ref, gw_ref, gb_ref, gms_ref, out_ref):
    out_ref[...] = _norm(z_ref[...], st_ref, gw_ref[...], gb_ref[...], gms_ref[...])


def _out_apply(z, st, gw, gb, gms):
    return pl.pallas_call(
        _k_out_apply,
        grid=(NB,),
        in_specs=[_rows(H), _full((8, H)),
                  _full((1, H)), _full((1, H)), _full((1, H))],
        out_specs=_rows(H),
        out_shape=jax.ShapeDtypeStruct((N, H), jnp.float32),
    )(z, st, gw, gb, gms)


def kernel(x, edge_index, edge_weight, mask, emb_table,
           egn_w, egn_b, egn_ms, mid_gn_w, mid_gn_b, mid_gn_ms,
           out_gn_w, out_gn_b, out_gn_ms,
           l1_Wt0, l1_Wt1, l1_bt0, l1_bt1, l1_gn_w, l1_gn_b, l1_gn_ms,
           l1_Wc0, l1_Wc1, l1_bc0, l1_bc1,
           l2_Wt0, l2_Wt1, l2_bt0, l2_bt1, l2_gn_w, l2_gn_b, l2_gn_ms,
           l2_Wc0, l2_Wc1, l2_bc0, l2_bc1):
    ids = x.reshape(N, 1).astype(jnp.int32)
    row = edge_index[0].astype(jnp.int32)
    col = edge_index[1].astype(jnp.int32)
    pad = E_PAD - E
    row_p = jnp.concatenate([row, jnp.full((pad,), N, jnp.int32)])
    col_p = jnp.concatenate([col, jnp.zeros((pad,), jnp.int32)])
    w_p = jnp.concatenate([edge_weight.astype(jnp.float32),
                           jnp.zeros((pad,), jnp.float32)])
    mf = mask.astype(jnp.float32)
    ed_p = (jnp.stack([col_p, row_p], axis=0)
            .reshape(2, NCHUNK, K).transpose(1, 0, 2))

    r1 = lambda a: a.reshape(1, H)

    deg = _sc_deg(row_p, w_p).reshape(N, 1)

    st0 = _emb_stats(ids, emb_table)
    h, xa = _emb_apply(ids, emb_table, st0, r1(egn_w), r1(egn_b), r1(egn_ms),
                       mf, l1_Wt0, l1_Wt1, r1(l1_bt0), r1(l1_bt1))

    agg1 = _sc_msg(xa, ed_p, w_p)
    st1 = _agg_stats(agg1, deg)
    h2, st2 = _conv_apply(agg1, deg, st1, r1(l1_gn_w), r1(l1_gn_b),
                          r1(l1_gn_ms), h, mf, l1_Wc0, l1_Wc1,
                          r1(l1_bc0), r1(l1_bc1))
    hp, xb = _mid_apply(h2, st2, r1(mid_gn_w), r1(mid_gn_b), r1(mid_gn_ms),
                        mf, l2_Wt0, l2_Wt1, r1(l2_bt0), r1(l2_bt1))

    agg2 = _sc_msg(xb, ed_p, w_p)
    st3 = _agg_stats(agg2, deg)
    zz, st4 = _conv_apply(agg2, deg, st3, r1(l2_gn_w), r1(l2_gn_b),
                          r1(l2_gn_ms), hp, mf, l2_Wc0, l2_Wc1,
                          r1(l2_bc0), r1(l2_bc1))
    return _out_apply(zz, st4, r1(out_gn_w), r1(out_gn_b), r1(out_gn_ms))


# speedup vs baseline: 4.2865x; 1.0828x over previous
import functools

import jax
import jax.numpy as jnp
from jax import lax
from jax.experimental import pallas as pl
from jax.experimental.pallas import tpu as pltpu
from jax.experimental.pallas import tpu_sc as plsc

N = 50000
E = 800000
H = 64
VOCAB = 64
Z = 0.8
EPS = 1e-5

NS = 16
K = 128
E_PAD = 802816
CH = E_PAD // (NS * K)
HALF = N // 2
ACC_ROWS = 26624
ZROWS = ACC_ROWS // NS
ZCH = ZROWS // K
OUT_REM = HALF - 15 * ZROWS
NCHUNK = E_PAD // K
MACC = 25088
MZROWS = MACC // NS
MZCH = MZROWS // K
MZREM = MZROWS - MZCH * K
MOCH15 = (HALF - 15 * MZROWS) // K
MOREM15 = HALF - 15 * MZROWS - MOCH15 * K

R_BLK = 2000
NB = N // R_BLK


def _sc_mesh():
    return plsc.VectorSubcoreMesh(core_axis_name="c", subcore_axis_name="s")


def _sc_deg(row_p, w_p):
    @functools.partial(
        pl.kernel,
        out_type=jax.ShapeDtypeStruct((N,), jnp.float32),
        mesh=_sc_mesh(),
        scratch_types=[
            pltpu.VMEM_SHARED((ACC_ROWS,), jnp.float32),
            pltpu.VMEM((K,), jnp.int32),
            pltpu.VMEM((K,), jnp.int32),
            pltpu.VMEM((K,), jnp.float32),
        ],
    )
    def k(row_hbm, w_hbm, out_hbm, accd, row_v, sidx_v, w_v):
        c = lax.axis_index("c")
        s = lax.axis_index("s")
        for j in range(K // 16):
            w_v[pl.ds(j * 16, 16)] = jnp.zeros((16,), jnp.float32)

        def zbody(t, carry):
            pltpu.sync_copy(w_v, accd.at[pl.ds(s * ZROWS + t * K, K)])
            return carry

        lax.fori_loop(0, ZCH, zbody, 0)
        plsc.subcore_barrier()
        base = c * HALF

        def body(ch, carry):
            e0 = (s * CH + ch) * K
            pltpu.sync_copy(row_hbm.at[pl.ds(e0, K)], row_v)
            pltpu.sync_copy(w_hbm.at[pl.ds(e0, K)], w_v)
            for j in range(K // 16):
                r = row_v[pl.ds(j * 16, 16)]
                loc = r - base
                ok = (loc >= 0) & (loc < HALF)
                sidx_v[pl.ds(j * 16, 16)] = jnp.where(ok, loc, HALF)
            pltpu.sync_copy(w_v, accd.at[sidx_v], add=True)
            return carry

        lax.fori_loop(0, CH, body, 0)
        plsc.subcore_barrier()
        ob = c * HALF

        @pl.when(s < NS - 1)
        def _():
            def obody(t, carry):
                off = s * ZROWS + t * K
                pltpu.sync_copy(accd.at[pl.ds(off, K)], w_v)
                pltpu.sync_copy(w_v, out_hbm.at[pl.ds(ob + off, K)])
                return carry

            lax.fori_loop(0, ZCH, obody, 0)

        @pl.when(s == NS - 1)
        def _():
            off = 15 * ZROWS
            pltpu.sync_copy(accd.at[pl.ds(off, OUT_REM)],
                            w_v.at[pl.ds(0, OUT_REM)])
            pltpu.sync_copy(w_v.at[pl.ds(0, OUT_REM)],
                            out_hbm.at[pl.ds(ob + off, OUT_REM)])

    return k(row_p, w_p)


def _sc_msg(x, col_p, row_p, w_p):
    @functools.partial(
        pl.kernel,
        out_type=jax.ShapeDtypeStruct((N, H), jnp.float32),
        mesh=_sc_mesh(),
        compiler_params=pltpu.CompilerParams(use_tc_tiling_on_sc=False),
        scratch_types=[
            pltpu.VMEM_SHARED((MACC, H), jnp.float32),
            pltpu.VMEM((K,), jnp.int32),
            pltpu.VMEM((K,), jnp.int32),
            pltpu.VMEM((K,), jnp.int32),
            pltpu.VMEM((K,), jnp.int32),
            pltpu.VMEM((K,), jnp.float32),
            pltpu.VMEM((K,), jnp.float32),
            pltpu.VMEM((1, K), jnp.int32),
            pltpu.VMEM((1, K), jnp.int32),
            pltpu.VMEM((K, H), jnp.float32),
            pltpu.VMEM((K, H), jnp.float32),
            pltpu.SemaphoreType.DMA,
            pltpu.SemaphoreType.DMA,
            pltpu.SemaphoreType.DMA,
            pltpu.SemaphoreType.DMA,
            pltpu.SemaphoreType.DMA,
            pltpu.SemaphoreType.DMA,
        ],
    )
    def k(x_hbm, col_hbm, row_hbm, w_hbm, out_hbm,
          acc, cb0, cb1, rw0, rw1, wb0, wb1, si0, si1, rb0, rb1,
          gs0, gs1, es0, es1, ss0, ss1):
        c = lax.axis_index("c")
        s = lax.axis_index("s")
        cb = (cb0, cb1)
        rw = (rw0, rw1)
        wb = (wb0, wb1)
        si = (si0, si1)
        rb = (rb0, rb1)
        gs = (gs0, gs1)
        es = (es0, es1)
        ss = (ss0, ss1)

        def zrbody(e, carry):
            for j in range(H // 16):
                rb0[e, pl.ds(j * 16, 16)] = jnp.zeros((16,), jnp.float32)
            return carry

        lax.fori_loop(0, K, zrbody, 0)

        def zbody(t, carry):
            pltpu.sync_copy(rb0, acc.at[pl.ds(s * MZROWS + t * K, K)])
            return carry

        lax.fori_loop(0, MZCH, zbody, 0)
        pltpu.sync_copy(rb0.at[pl.ds(0, MZREM)],
                        acc.at[pl.ds(s * MZROWS + MZCH * K, MZREM)])
        plsc.subcore_barrier()
        base = c * HALF
        e00 = s * CH * K

        def compute_sidx(p):
            for g in range(K // 16):
                r = rw[p][pl.ds(g * 16, 16)]
                loc = r - base
                ok = (loc >= 0) & (loc < HALF)
                si[p][0, pl.ds(g * 16, 16)] = jnp.where(ok, loc, 0)
                w16 = wb[p][pl.ds(g * 16, 16)]
                wb[p][pl.ds(g * 16, 16)] = jnp.where(
                    ok, w16, jnp.zeros((16,), jnp.float32))

        def start_gather(p, sem):
            pltpu.async_copy(x_hbm.at[cb[p]], rb[p], sem)

        def drain_gather(p, sem):
            pltpu.make_async_copy(x_hbm.at[cb[p]], rb[p], sem).wait()

        def start_edload(S, p, sem):
            e0 = e00 + S * K
            pltpu.async_copy(col_hbm.at[pl.ds(e0, K)], cb[p], sem)
            pltpu.async_copy(row_hbm.at[pl.ds(e0, K)], rw[p], sem)
            pltpu.async_copy(w_hbm.at[pl.ds(e0, K)], wb[p], sem)

        def drain_edload(S, p, sem):
            e0 = e00 + S * K
            pltpu.make_async_copy(col_hbm.at[pl.ds(e0, K)], cb[p], sem).wait()
            pltpu.make_async_copy(row_hbm.at[pl.ds(e0, K)], rw[p], sem).wait()
            pltpu.make_async_copy(w_hbm.at[pl.ds(e0, K)], wb[p], sem).wait()

        def start_scatter(p, sem):
            pltpu.async_copy(rb[p], acc.at[si[p].at[0]], sem, add=True)

        def drain_scatter(p, sem):
            pltpu.make_async_copy(rb[p], acc.at[si[p].at[0]], sem).wait()

        def scale(p):
            def sgrp(g, cc):
                wv16 = wb[p][pl.ds(g * 16, 16)]
                for l in range(16):
                    wl = wv16[l]
                    e = g * 16 + l
                    for q in range(H // 16):
                        rb[p][e, pl.ds(q * 16, 16)] = (
                            rb[p][e, pl.ds(q * 16, 16)] * wl)
                return cc

            lax.fori_loop(0, K // 16, sgrp, 0)

        start_edload(0, 0, es0)
        drain_edload(0, 0, es0)
        compute_sidx(0)
        start_gather(0, gs0)
        start_edload(1, 1, es1)

        def phase(S, p):
            @pl.when(S < CH - 1)
            def _():
                drain_edload(S + 1, 1 - p, es[1 - p])

            @pl.when(S >= 1)
            def _():
                drain_scatter(1 - p, ss[1 - p])

            @pl.when(S < CH - 1)
            def _():
                compute_sidx(1 - p)

            drain_gather(p, gs[p])

            @pl.when(S < CH - 1)
            def _():
                start_gather(1 - p, gs[1 - p])

            scale(p)
            start_scatter(p, ss[p])

            @pl.when(S < CH - 2)
            def _():
                start_edload(S + 2, p, es[p])

        def body(t, carry):
            phase(2 * t, 0)
            phase(2 * t + 1, 1)
            return carry

        lax.fori_loop(0, CH // 2, body, 0)
        drain_scatter(1, ss1)
        plsc.subcore_barrier()
        ob = c * HALF

        def ocopy(off, rows):
            pltpu.sync_copy(acc.at[pl.ds(off, rows)],
                            rb0.at[pl.ds(0, rows)])
            pltpu.sync_copy(rb0.at[pl.ds(0, rows)],
                            out_hbm.at[pl.ds(ob + off, rows)])

        @pl.when(s < NS - 1)
        def _():
            def obody(t, carry):
                ocopy(s * MZROWS + t * K, K)
                return carry

            lax.fori_loop(0, MZCH, obody, 0)
            ocopy(s * MZROWS + MZCH * K, MZREM)

        @pl.when(s == NS - 1)
        def _():
            def obody(t, carry):
                ocopy(15 * MZROWS + t * K, K)
                return carry

            lax.fori_loop(0, MOCH15, obody, 0)
            ocopy(15 * MZROWS + MOCH15 * K, MOREM15)

    return k(x, col_p, row_p, w_p)


def _rows(w):
    return pl.BlockSpec((R_BLK, w), lambda i: (i, 0))


def _full(shape):
    nd = len(shape)
    return pl.BlockSpec(shape, lambda i: (0,) * nd)


def _stats_update(out_ref, vals):
    i = pl.program_id(0)
    sm = jnp.sum(vals, axis=0, keepdims=True)
    sq = jnp.sum(vals * vals, axis=0, keepdims=True)
    upd = jnp.concatenate(
        [sm, sq, jnp.zeros((6, H), jnp.float32)], axis=0)

    @pl.when(i == 0)
    def _():
        out_ref[...] = jnp.zeros_like(out_ref)

    out_ref[...] += upd


def _norm(xv, st_ref, w, b, ms):
    st = st_ref[...]
    mean = st[0:1, :] * (1.0 / N)
    ex2 = st[1:2, :] * (1.0 / N)
    var = ex2 - (2.0 - ms) * ms * mean * mean
    return w * (xv - ms * mean) * lax.rsqrt(var + EPS) + b


def _blend(x0, x1, mf):
    return mf * (Z * x1 + (1.0 - Z) * x0) + (1.0 - mf) * (Z * x0 + (1.0 - Z) * x1)


def _onehot_emb(ids_ref, emb_ref):
    ids = ids_ref[...]
    oh = (ids == lax.broadcasted_iota(jnp.int32, (R_BLK, VOCAB), 1))
    return jnp.dot(oh.astype(jnp.float32), emb_ref[...],
                   preferred_element_type=jnp.float32)


def _k_emb_stats(ids_ref, emb_ref, out_ref):
    _stats_update(out_ref, _onehot_emb(ids_ref, emb_ref))


def _emb_stats(ids, emb):
    return pl.pallas_call(
        _k_emb_stats,
        grid=(NB,),
        in_specs=[_rows(1), _full((VOCAB, H))],
        out_specs=_full((8, H)),
        out_shape=jax.ShapeDtypeStruct((8, H), jnp.float32),
    )(ids, emb)


def _k_emb_apply(ids_ref, emb_ref, st_ref, gw_ref, gb_ref, gms_ref, mf_ref,
                 wt0_ref, wt1_ref, bt0_ref, bt1_ref, h_ref, xa_ref):
    h0 = _onehot_emb(ids_ref, emb_ref)
    h = _norm(h0, st_ref, gw_ref[...], gb_ref[...], gms_ref[...])
    mf = mf_ref[...]
    x1 = jnp.maximum(jnp.dot(h, wt1_ref[...],
                             preferred_element_type=jnp.float32) + bt1_ref[...], 0.0)
    x0 = jnp.maximum(jnp.dot(h, wt0_ref[...],
                             preferred_element_type=jnp.float32) + bt0_ref[...], 0.0)
    h_ref[...] = h
    xa_ref[...] = _blend(x0, x1, mf)


def _emb_apply(ids, emb, st, gw, gb, gms, mf, wt0, wt1, bt0, bt1):
    return pl.pallas_call(
        _k_emb_apply,
        grid=(NB,),
        in_specs=[_rows(1), _full((VOCAB, H)), _full((8, H)),
                  _full((1, H)), _full((1, H)), _full((1, H)), _rows(1),
                  _full((H, H)), _full((H, H)), _full((1, H)), _full((1, H))],
        out_specs=[_rows(H), _rows(H)],
        out_shape=[jax.ShapeDtypeStruct((N, H), jnp.float32),
                   jax.ShapeDtypeStruct((N, H), jnp.float32)],
    )(ids, emb, st, gw, gb, gms, mf, wt0, wt1, bt0, bt1)


def _k_agg_stats(agg_ref, deg_ref, out_ref):
    deg = deg_ref[...]
    degf = jnp.where(deg < 0.5, deg + 1.0, deg)
    _stats_update(out_ref, agg_ref[...] / degf)


def _agg_stats(agg, deg):
    return pl.pallas_call(
        _k_agg_stats,
        grid=(NB,),
        in_specs=[_rows(H), _rows(1)],
        out_specs=_full((8, H)),
        out_shape=jax.ShapeDtypeStruct((8, H), jnp.float32),
    )(agg, deg)


def _k_conv_apply(agg_ref, deg_ref, st_ref, gw_ref, gb_ref, gms_ref,
                  hprev_ref, mf_ref, wc0_ref, wc1_ref, bc0_ref, bc1_ref,
                  h2_ref, st2_ref):
    deg = deg_ref[...]
    degf = jnp.where(deg < 0.5, deg + 1.0, deg)
    y = agg_ref[...] / degf
    yn = _norm(y, st_ref, gw_ref[...], gb_ref[...], gms_ref[...])
    cat = jnp.concatenate([yn, hprev_ref[...]], axis=1)
    c1 = jnp.dot(cat, wc1_ref[...], preferred_element_type=jnp.float32) + bc1_ref[...]
    c0 = jnp.dot(cat, wc0_ref[...], preferred_element_type=jnp.float32) + bc0_ref[...]
    h2 = _blend(c0, c1, mf_ref[...])
    h2_ref[...] = h2
    _stats_update(st2_ref, h2)


def _conv_apply(agg, deg, st, gw, gb, gms, hprev, mf, wc0, wc1, bc0, bc1):
    return pl.pallas_call(
        _k_conv_apply,
        grid=(NB,),
        in_specs=[_rows(H), _rows(1), _full((8, H)),
                  _full((1, H)), _full((1, H)), _full((1, H)),
                  _rows(H), _rows(1),
                  _full((2 * H, H)), _full((2 * H, H)),
                  _full((1, H)), _full((1, H))],
        out_specs=[_rows(H), _full((8, H))],
        out_shape=[jax.ShapeDtypeStruct((N, H), jnp.float32),
                   jax.ShapeDtypeStruct((8, H), jnp.float32)],
    )(agg, deg, st, gw, gb, gms, hprev, mf, wc0, wc1, bc0, bc1)


def _k_mid_apply(h2_ref, st_ref, gw_ref, gb_ref, gms_ref, mf_ref,
                 wt0_ref, wt1_ref, bt0_ref, bt1_ref, hp_ref, xb_ref):
    hp = jnp.maximum(
        _norm(h2_ref[...], st_ref, gw_ref[...], gb_ref[...], gms_ref[...]), 0.0)
    mf = mf_ref[...]
    x1 = jnp.maximum(jnp.dot(hp, wt1_ref[...],
                             preferred_element_type=jnp.float32) + bt1_ref[...], 0.0)
    x0 = jnp.maximum(jnp.dot(hp, wt0_ref[...],
                             preferred_element_type=jnp.float32) + bt0_ref[...], 0.0)
    hp_ref[...] = hp
    xb_ref[...] = _blend(x0, x1, mf)


def _mid_apply(h2, st, gw, gb, gms, mf, wt0, wt1, bt0, bt1):
    return pl.pallas_call(
        _k_mid_apply,
        grid=(NB,),
        in_specs=[_rows(H), _full((8, H)),
                  _full((1, H)), _full((1, H)), _full((1, H)), _rows(1),
                  _full((H, H)), _full((H, H)), _full((1, H)), _full((1, H))],
        out_specs=[_rows(H), _rows(H)],
        out_shape=[jax.ShapeDtypeStruct((N, H), jnp.float32),
                   jax.ShapeDtypeStruct((N, H), jnp.float32)],
    )(h2, st, gw, gb, gms, mf, wt0, wt1, bt0, bt1)


def _k_out_apply(z_ref, st_ref, gw_ref, gb_ref, gms_ref, out_ref):
    out_ref[...] = _norm(z_ref[...], st_ref, gw_ref[...], gb_ref[...], gms_ref[...])


def _out_apply(z, st, gw, gb, gms):
    return pl.pallas_call(
        _k_out_apply,
        grid=(NB,),
        in_specs=[_rows(H), _full((8, H)),
                  _full((1, H)), _full((1, H)), _full((1, H))],
        out_specs=_rows(H),
        out_shape=jax.ShapeDtypeStruct((N, H), jnp.float32),
    )(z, st, gw, gb, gms)


def kernel(x, edge_index, edge_weight, mask, emb_table,
           egn_w, egn_b, egn_ms, mid_gn_w, mid_gn_b, mid_gn_ms,
           out_gn_w, out_gn_b, out_gn_ms,
           l1_Wt0, l1_Wt1, l1_bt0, l1_bt1, l1_gn_w, l1_gn_b, l1_gn_ms,
           l1_Wc0, l1_Wc1, l1_bc0, l1_bc1,
           l2_Wt0, l2_Wt1, l2_bt0, l2_bt1, l2_gn_w, l2_gn_b, l2_gn_ms,
           l2_Wc0, l2_Wc1, l2_bc0, l2_bc1):
    ids = x.reshape(N, 1).astype(jnp.int32)
    row = edge_index[0].astype(jnp.int32)
    col = edge_index[1].astype(jnp.int32)
    pad = E_PAD - E
    row_p = jnp.concatenate([row, jnp.full((pad,), N, jnp.int32)])
    col_p = jnp.concatenate([col, jnp.zeros((pad,), jnp.int32)])
    w_p = jnp.concatenate([edge_weight.astype(jnp.float32),
                           jnp.zeros((pad,), jnp.float32)])
    mf = mask.astype(jnp.float32)

    r1 = lambda a: a.reshape(1, H)

    deg = _sc_deg(row_p, w_p).reshape(N, 1)

    st0 = _emb_stats(ids, emb_table)
    h, xa = _emb_apply(ids, emb_table, st0, r1(egn_w), r1(egn_b), r1(egn_ms),
                       mf, l1_Wt0, l1_Wt1, r1(l1_bt0), r1(l1_bt1))

    agg1 = _sc_msg(xa, col_p, row_p, w_p)
    st1 = _agg_stats(agg1, deg)
    h2, st2 = _conv_apply(agg1, deg, st1, r1(l1_gn_w), r1(l1_gn_b),
                          r1(l1_gn_ms), h, mf, l1_Wc0, l1_Wc1,
                          r1(l1_bc0), r1(l1_bc1))
    hp, xb = _mid_apply(h2, st2, r1(mid_gn_w), r1(mid_gn_b), r1(mid_gn_ms),
                        mf, l2_Wt0, l2_Wt1, r1(l2_bt0), r1(l2_bt1))

    agg2 = _sc_msg(xb, col_p, row_p, w_p)
    st3 = _agg_stats(agg2, deg)
    zz, st4 = _conv_apply(agg2, deg, st3, r1(l2_gn_w), r1(l2_gn_b),
                          r1(l2_gn_ms), hp, mf, l2_Wc0, l2_Wc1,
                          r1(l2_bc0), r1(l2_bc1))
    return _out_apply(zz, st4, r1(out_gn_w), r1(out_gn_b), r1(out_gn_ms))
